# baseline (speedup 1.0000x reference)
import functools

import jax
import jax.numpy as jnp
from jax import lax
from jax.experimental import pallas as pl
from jax.experimental.pallas import tpu as pltpu
from jax.experimental.pallas import tpu_sc as plsc

N = 10000
E = 320000
D = 128
H = 128

NC = 2
NS = 16
LANES = 16
HW = H // NC

EB = 64
NROWS = E // EB
NROWS_PAD = 5120
R1T = NROWS_PAD // NS
NP = 10240
NSEG = NP // NS
HT = 624


def _pre_body(s_ref, t_ref, w_ref, bl_ref, w1_ref, w2_ref, ba_ref,
              th_ref, as_ref, at_ref):
    w = w_ref[...]
    dn = (((1,), (1,)), ((), ()))
    sh = lax.dot_general(s_ref[...], w, dn,
                         preferred_element_type=jnp.float32) + bl_ref[...]
    th = lax.dot_general(t_ref[...], w, dn,
                         preferred_element_type=jnp.float32) + bl_ref[...]
    th_ref[0] = th[:, :HW]
    th_ref[1] = th[:, HW:]
    as_ref[...] = sh @ w1_ref[...] + ba_ref[0, 0]
    at_ref[...] = th @ w2_ref[...]


def _pre(source_h, target_h, W, b_lin, w1, w2, b_att):
    blk = 1000
    grid = N // blk
    return pl.pallas_call(
        _pre_body,
        grid=(grid,),
        in_specs=[
            pl.BlockSpec((blk, D), lambda i: (i, 0)),
            pl.BlockSpec((blk, D), lambda i: (i, 0)),
            pl.BlockSpec((H, D), lambda i: (0, 0)),
            pl.BlockSpec((1, H), lambda i: (0, 0)),
            pl.BlockSpec((H, 1), lambda i: (0, 0)),
            pl.BlockSpec((H, 1), lambda i: (0, 0)),
            pl.BlockSpec((1, 1), lambda i: (0, 0)),
        ],
        out_specs=[
            pl.BlockSpec((2, blk, HW), lambda i: (0, i, 0)),
            pl.BlockSpec((blk, 1), lambda i: (i, 0)),
            pl.BlockSpec((blk, 1), lambda i: (i, 0)),
        ],
        out_shape=[
            jax.ShapeDtypeStruct((2, N, HW), jnp.float32),
            jax.ShapeDtypeStruct((N, 1), jnp.float32),
            jax.ShapeDtypeStruct((N, 1), jnp.float32),
        ],
    )(source_h, target_h, W, b_lin, w1, w2, b_att)


def _post_body(p_ref, b_ref, o_ref):
    o_ref[:, :HW] = p_ref[0] + b_ref[:, :HW]
    o_ref[:, HW:] = p_ref[1] + b_ref[:, HW:]


def _post(partials, bias):
    blk = 1000
    return pl.pallas_call(
        _post_body,
        grid=(N // blk,),
        in_specs=[
            pl.BlockSpec((2, blk, HW), lambda i: (0, i, 0)),
            pl.BlockSpec((1, H), lambda i: (0, 0)),
        ],
        out_specs=pl.BlockSpec((blk, H), lambda i: (i, 0)),
        out_shape=jax.ShapeDtypeStruct((N, H), jnp.float32),
    )(partials, bias)


def _edge_exp(a, b):
    x = a + b
    t = jnp.exp(-2.0 * jnp.abs(x))
    th = (1.0 - t) / (1.0 + t)
    e = jnp.where(x < 0.0, -th, th)
    return jnp.exp(e)


def _sc_body(src_hbm, tgt_hbm, as_hbm, at_hbm, th2_hbm, out_hbm,
             asv, atv, dtile, srcv, tgtv, rows, attnb, didx,
             dacc, hp, gs0, gs1, gs2, gs3, ss0, ss1, ss2, ss3):
    gsem = [gs0, gs1, gs2, gs3]
    ssem = [ss0, ss1, ss2, ss3]
    c = lax.axis_index("c")
    s = lax.axis_index("s")
    thc = th2_hbm.at[c]

    pltpu.sync_copy(as_hbm, asv)
    pltpu.sync_copy(at_hbm, atv)

    zero = jnp.zeros((LANES,), jnp.float32)
    lanes_iota = lax.iota(jnp.int32, LANES)

    def zero_dtile(i, _):
        dtile[i] = zero
        return 0

    lax.fori_loop(0, NP // LANES, zero_dtile, 0)
    for k in range(NP // LANES // EB):
        for j in range(EB // LANES):
            didx[k, pl.ds(j * LANES, LANES)] = (
                lanes_iota + (k * EB + j * LANES))

    off = s * (NSEG // LANES)
    pltpu.sync_copy(dtile.at[pl.ds(0, NSEG // LANES)],
                    dacc.at[pl.ds(off, NSEG // LANES)])

    def zero_rows(i, _):
        for j in range(HW // LANES):
            rows[0, i, pl.ds(j * LANES, LANES)] = zero
        return 0

    lax.fori_loop(0, EB, zero_rows, 0)
    hbase = s * HT
    for k in range(HT // EB):
        pltpu.sync_copy(rows.at[0], hp.at[pl.ds(hbase + k * EB, EB)])
    rem = HT - (HT // EB) * EB
    pltpu.sync_copy(rows.at[0, pl.ds(0, rem)],
                    hp.at[pl.ds(hbase + (HT // EB) * EB, rem)])

    @pl.when(s == NS - 1)
    def _():
        pltpu.sync_copy(rows.at[0, pl.ds(0, 16)],
                        hp.at[pl.ds(NS * HT, 16)])

    pltpu.sync_copy(src_hbm.at[pl.ds(s * R1T, R1T)], srcv)
    pltpu.sync_copy(tgt_hbm.at[pl.ds(s * R1T, R1T)], tgtv)
    cnt = jnp.minimum(R1T, NROWS - s * R1T)

    def p1_row(r, _):
        for j in range(EB // LANES):
            s16 = srcv[r, pl.ds(j * LANES, LANES)]
            t16 = tgtv[r, pl.ds(j * LANES, LANES)]
            ex = _edge_exp(plsc.load_gather(asv, [s16]),
                           plsc.load_gather(atv, [t16]))
            plsc.addupdate_scatter(
                dtile,
                [lax.shift_right_logical(s16, 4),
                 lax.bitwise_and(s16, 15)], ex)
        return 0

    lax.fori_loop(0, cnt, p1_row, 0)

    plsc.subcore_barrier()
    for k in range(NP // LANES // EB):
        pltpu.sync_copy(dtile.at[pl.ds(k * EB, EB)],
                        dacc.at[didx.at[k]], add=True)
    plsc.subcore_barrier()

    pltpu.sync_copy(dacc, dtile)

    def issue(g, buf):
        pltpu.async_copy(thc.at[tgtv.at[g]], rows.at[buf], gsem[buf])

    def wait_gather(buf):
        pltpu.make_async_copy(thc.at[tgtv.at[0]], rows.at[buf],
                              gsem[buf]).wait()

    def scatter(g, buf):
        pltpu.async_copy(rows.at[buf], hp.at[srcv.at[g]], ssem[buf],
                         add=True)

    def wait_scatter(buf):
        pltpu.make_async_copy(rows.at[buf], hp.at[srcv.at[0]],
                              ssem[buf]).wait()

    def consume(g, buf):
        def chunk(jc, _):
            base = jc * LANES
            s16 = srcv[g, pl.ds(base, LANES)]
            t16 = tgtv[g, pl.ds(base, LANES)]
            ex = _edge_exp(plsc.load_gather(asv, [s16]),
                           plsc.load_gather(atv, [t16]))
            d = plsc.load_gather(
                dtile,
                [lax.shift_right_logical(s16, 4),
                 lax.bitwise_and(s16, 15)])
            attn16 = ex / d
            for l in range(LANES):
                a = attn16[l]
                i = base + l
                for j in range(HW // LANES):
                    sl = pl.ds(j * LANES, LANES)
                    rows[buf, i, sl] = rows[buf, i, sl] * a
            return 0

        lax.fori_loop(0, EB // LANES, chunk, 0)

    for b in range(4):
        issue(b, b)

    def p2_quad(i, _):
        for b in range(4):
            g = 4 * i + b
            wait_gather(b)
            consume(g, b)
            sc = (b + 3) % 4
            gp = g - 1

            @pl.when(gp >= 0)
            def _():
                wait_scatter(sc)

                @pl.when(gp + 4 < cnt)
                def _():
                    issue(gp + 4, sc)

            scatter(g, b)
        return 0

    lax.fori_loop(0, cnt // 4, p2_quad, 0)
    wait_scatter(3)

    plsc.subcore_barrier()
    pltpu.sync_copy(hp.at[pl.ds(hbase, HT)],
                    out_hbm.at[c, pl.ds(hbase, HT)])

    @pl.when(s == NS - 1)
    def _():
        pltpu.sync_copy(hp.at[pl.ds(NS * HT, 16)],
                        out_hbm.at[c, pl.ds(NS * HT, 16)])


_sc_edge = functools.partial(
    pl.kernel,
    out_type=jax.ShapeDtypeStruct((NC, N, HW), jnp.float32),
    mesh=plsc.VectorSubcoreMesh(core_axis_name="c", subcore_axis_name="s"),
    compiler_params=pltpu.CompilerParams(needs_layout_passes=False,
                                         use_tc_tiling_on_sc=False),
    scratch_types=[
        pltpu.VMEM((N,), jnp.float32),
        pltpu.VMEM((N,), jnp.float32),
        pltpu.VMEM((NP // LANES, LANES), jnp.float32),
        pltpu.VMEM((R1T, EB), jnp.int32),
        pltpu.VMEM((R1T, EB), jnp.int32),
        pltpu.VMEM((4, EB, HW), jnp.float32),
        pltpu.VMEM((EB,), jnp.float32),
        pltpu.VMEM((NP // LANES // EB, EB), jnp.int32),
        pltpu.VMEM_SHARED((NP // LANES, LANES), jnp.float32),
        pltpu.VMEM_SHARED((N, HW), jnp.float32),
        pltpu.SemaphoreType.DMA,
        pltpu.SemaphoreType.DMA,
        pltpu.SemaphoreType.DMA,
        pltpu.SemaphoreType.DMA,
        pltpu.SemaphoreType.DMA,
        pltpu.SemaphoreType.DMA,
        pltpu.SemaphoreType.DMA,
        pltpu.SemaphoreType.DMA,
    ],
)(_sc_body)


def kernel(source_h, target_h, edge_list, W, b_lin, W_att, b_att, bias):
    w1 = W_att[0, :H].reshape(H, 1).astype(jnp.float32)
    w2 = W_att[0, H:].reshape(H, 1).astype(jnp.float32)
    th2, a_s, a_t = _pre(source_h, target_h, W, b_lin.reshape(1, H),
                         w1, w2, b_att.reshape(1, 1))
    pad = NROWS_PAD * EB - E
    src2d = jnp.pad(edge_list[0].astype(jnp.int32),
                    (0, pad)).reshape(NROWS_PAD, EB)
    tgt2d = jnp.pad(edge_list[1].astype(jnp.int32),
                    (0, pad)).reshape(NROWS_PAD, EB)
    partials = _sc_edge(src2d, tgt2d, a_s.reshape(N), a_t.reshape(N), th2)
    return _post(partials, bias.reshape(1, H))


# speedup vs baseline: 1.0675x; 1.0675x over previous
import functools

import jax
import jax.numpy as jnp
from jax import lax
from jax.experimental import pallas as pl
from jax.experimental.pallas import tpu as pltpu
from jax.experimental.pallas import tpu_sc as plsc

N = 10000
E = 320000
D = 128
H = 128

NC = 2
NS = 16
LANES = 16
HW = H // NC

EB = 64
NROWS = E // EB
NROWS_PAD = 5120
R1T = NROWS_PAD // NS
NP = 10240
NSEG = NP // NS
HT = 624


def _pre_body(s_ref, t_ref, w_ref, bl_ref, w1_ref, w2_ref, ba_ref,
              th_ref, as_ref, at_ref):
    w = w_ref[...]
    dn = (((1,), (1,)), ((), ()))
    sh = lax.dot_general(s_ref[...], w, dn,
                         preferred_element_type=jnp.float32) + bl_ref[...]
    th = lax.dot_general(t_ref[...], w, dn,
                         preferred_element_type=jnp.float32) + bl_ref[...]
    th_ref[0] = th[:, :HW]
    th_ref[1] = th[:, HW:]
    as_ref[...] = sh @ w1_ref[...] + ba_ref[0, 0]
    at_ref[...] = th @ w2_ref[...]


def _pre(source_h, target_h, W, b_lin, w1, w2, b_att):
    blk = 1000
    grid = N // blk
    return pl.pallas_call(
        _pre_body,
        grid=(grid,),
        in_specs=[
            pl.BlockSpec((blk, D), lambda i: (i, 0)),
            pl.BlockSpec((blk, D), lambda i: (i, 0)),
            pl.BlockSpec((H, D), lambda i: (0, 0)),
            pl.BlockSpec((1, H), lambda i: (0, 0)),
            pl.BlockSpec((H, 1), lambda i: (0, 0)),
            pl.BlockSpec((H, 1), lambda i: (0, 0)),
            pl.BlockSpec((1, 1), lambda i: (0, 0)),
        ],
        out_specs=[
            pl.BlockSpec((2, blk, HW), lambda i: (0, i, 0)),
            pl.BlockSpec((blk, 1), lambda i: (i, 0)),
            pl.BlockSpec((blk, 1), lambda i: (i, 0)),
        ],
        out_shape=[
            jax.ShapeDtypeStruct((2, N, HW), jnp.float32),
            jax.ShapeDtypeStruct((N, 1), jnp.float32),
            jax.ShapeDtypeStruct((N, 1), jnp.float32),
        ],
    )(source_h, target_h, W, b_lin, w1, w2, b_att)


def _post_body(p_ref, b_ref, o_ref):
    o_ref[:, :HW] = p_ref[0] + b_ref[:, :HW]
    o_ref[:, HW:] = p_ref[1] + b_ref[:, HW:]


def _post(partials, bias):
    blk = 1000
    return pl.pallas_call(
        _post_body,
        grid=(N // blk,),
        in_specs=[
            pl.BlockSpec((2, blk, HW), lambda i: (0, i, 0)),
            pl.BlockSpec((1, H), lambda i: (0, 0)),
        ],
        out_specs=pl.BlockSpec((blk, H), lambda i: (i, 0)),
        out_shape=jax.ShapeDtypeStruct((N, H), jnp.float32),
    )(partials, bias)


def _edge_exp(a, b):
    x = a + b
    t = jnp.exp(-2.0 * jnp.abs(x))
    th = (1.0 - t) / (1.0 + t)
    e = jnp.where(x < 0.0, -th, th)
    return jnp.exp(e)


def _sc_body(src_hbm, tgt_hbm, as_hbm, at_hbm, th2_hbm, bias_hbm, out_hbm,
             asv, atv, dtile, srcv, tgtv, rows, biasv, didx,
             dacc, hp, gs0, gs1, gs2, gs3, ss0, ss1, ss2, ss3):
    gsem = [gs0, gs1, gs2, gs3]
    ssem = [ss0, ss1, ss2, ss3]
    c = lax.axis_index("c")
    s = lax.axis_index("s")
    thc = th2_hbm.at[c]

    pltpu.sync_copy(as_hbm, asv)
    pltpu.sync_copy(at_hbm, atv)
    pltpu.sync_copy(bias_hbm.at[c], biasv)

    zero = jnp.zeros((LANES,), jnp.float32)
    lanes_iota = lax.iota(jnp.int32, LANES)

    def zero_dtile(i, _):
        dtile[i] = zero
        return 0

    lax.fori_loop(0, NP // LANES, zero_dtile, 0)
    for k in range(NP // LANES // EB):
        for j in range(EB // LANES):
            didx[k, pl.ds(j * LANES, LANES)] = (
                lanes_iota + (k * EB + j * LANES))

    off = s * (NSEG // LANES)
    pltpu.sync_copy(dtile.at[pl.ds(0, NSEG // LANES)],
                    dacc.at[pl.ds(off, NSEG // LANES)])

    def bias_rows(i, _):
        for j in range(HW // LANES):
            sl = pl.ds(j * LANES, LANES)
            rows[0, i, sl] = biasv[sl]
        return 0

    lax.fori_loop(0, EB, bias_rows, 0)
    hbase = s * HT
    for k in range(HT // EB):
        pltpu.sync_copy(rows.at[0], hp.at[pl.ds(hbase + k * EB, EB)])
    rem = HT - (HT // EB) * EB
    pltpu.sync_copy(rows.at[0, pl.ds(0, rem)],
                    hp.at[pl.ds(hbase + (HT // EB) * EB, rem)])

    @pl.when(s == NS - 1)
    def _():
        pltpu.sync_copy(rows.at[0, pl.ds(0, 16)],
                        hp.at[pl.ds(NS * HT, 16)])

    cnt = jnp.minimum(R1T, NROWS - s * R1T)

    @pl.when(s < NS - 1)
    def _():
        pltpu.sync_copy(src_hbm.at[pl.ds(s * R1T, R1T)], srcv)
        pltpu.sync_copy(tgt_hbm.at[pl.ds(s * R1T, R1T)], tgtv)

    @pl.when(s == NS - 1)
    def _():
        lastn = NROWS - (NS - 1) * R1T
        pltpu.sync_copy(src_hbm.at[pl.ds((NS - 1) * R1T, lastn)],
                        srcv.at[pl.ds(0, lastn)])
        pltpu.sync_copy(tgt_hbm.at[pl.ds((NS - 1) * R1T, lastn)],
                        tgtv.at[pl.ds(0, lastn)])

    def p1_row(r, _):
        for j in range(EB // LANES):
            s16 = srcv[r, pl.ds(j * LANES, LANES)]
            t16 = tgtv[r, pl.ds(j * LANES, LANES)]
            ex = _edge_exp(plsc.load_gather(asv, [s16]),
                           plsc.load_gather(atv, [t16]))
            plsc.addupdate_scatter(
                dtile,
                [lax.shift_right_logical(s16, 4),
                 lax.bitwise_and(s16, 15)], ex)
        return 0

    lax.fori_loop(0, cnt, p1_row, 0)

    plsc.subcore_barrier()
    for k in range(NP // LANES // EB):
        pltpu.sync_copy(dtile.at[pl.ds(k * EB, EB)],
                        dacc.at[didx.at[k]], add=True)
    plsc.subcore_barrier()

    pltpu.sync_copy(dacc, dtile)

    def issue(g, buf):
        pltpu.async_copy(thc.at[tgtv.at[g]], rows.at[buf], gsem[buf])

    def wait_gather(buf):
        pltpu.make_async_copy(thc.at[tgtv.at[0]], rows.at[buf],
                              gsem[buf]).wait()

    def scatter(g, buf):
        pltpu.async_copy(rows.at[buf], hp.at[srcv.at[g]], ssem[buf],
                         add=True)

    def wait_scatter(buf):
        pltpu.make_async_copy(rows.at[buf], hp.at[srcv.at[0]],
                              ssem[buf]).wait()

    def consume(g, buf):
        def chunk(jc, _):
            base = jc * LANES
            s16 = srcv[g, pl.ds(base, LANES)]
            t16 = tgtv[g, pl.ds(base, LANES)]
            ex = _edge_exp(plsc.load_gather(asv, [s16]),
                           plsc.load_gather(atv, [t16]))
            d = plsc.load_gather(
                dtile,
                [lax.shift_right_logical(s16, 4),
                 lax.bitwise_and(s16, 15)])
            attn16 = ex / d
            for l in range(LANES):
                a = attn16[l]
                i = base + l
                for j in range(HW // LANES):
                    sl = pl.ds(j * LANES, LANES)
                    rows[buf, i, sl] = rows[buf, i, sl] * a
            return 0

        lax.fori_loop(0, EB // LANES, chunk, 0)

    for b in range(4):
        issue(b, b)

    def p2_quad(i, _):
        for b in range(4):
            g = 4 * i + b
            wait_gather(b)
            consume(g, b)
            sc = (b + 3) % 4
            gp = g - 1

            @pl.when(gp >= 0)
            def _():
                wait_scatter(sc)

                @pl.when(gp + 4 < cnt)
                def _():
                    issue(gp + 4, sc)

            scatter(g, b)
        return 0

    lax.fori_loop(0, cnt // 4, p2_quad, 0)
    wait_scatter(3)

    plsc.subcore_barrier()
    col = c * HW
    pltpu.sync_copy(hp.at[pl.ds(hbase, HT)],
                    out_hbm.at[pl.ds(hbase, HT), pl.ds(col, HW)])

    @pl.when(s == NS - 1)
    def _():
        pltpu.sync_copy(hp.at[pl.ds(NS * HT, 16)],
                        out_hbm.at[pl.ds(NS * HT, 16), pl.ds(col, HW)])


_sc_edge = functools.partial(
    pl.kernel,
    out_type=jax.ShapeDtypeStruct((N, H), jnp.float32),
    mesh=plsc.VectorSubcoreMesh(core_axis_name="c", subcore_axis_name="s"),
    compiler_params=pltpu.CompilerParams(needs_layout_passes=False,
                                         use_tc_tiling_on_sc=False),
    scratch_types=[
        pltpu.VMEM((N,), jnp.float32),
        pltpu.VMEM((N,), jnp.float32),
        pltpu.VMEM((NP // LANES, LANES), jnp.float32),
        pltpu.VMEM((R1T, EB), jnp.int32),
        pltpu.VMEM((R1T, EB), jnp.int32),
        pltpu.VMEM((4, EB, HW), jnp.float32),
        pltpu.VMEM((HW,), jnp.float32),
        pltpu.VMEM((NP // LANES // EB, EB), jnp.int32),
        pltpu.VMEM_SHARED((NP // LANES, LANES), jnp.float32),
        pltpu.VMEM_SHARED((N, HW), jnp.float32),
        pltpu.SemaphoreType.DMA,
        pltpu.SemaphoreType.DMA,
        pltpu.SemaphoreType.DMA,
        pltpu.SemaphoreType.DMA,
        pltpu.SemaphoreType.DMA,
        pltpu.SemaphoreType.DMA,
        pltpu.SemaphoreType.DMA,
        pltpu.SemaphoreType.DMA,
    ],
)(_sc_body)


def kernel(source_h, target_h, edge_list, W, b_lin, W_att, b_att, bias):
    w1 = W_att[0, :H].reshape(H, 1).astype(jnp.float32)
    w2 = W_att[0, H:].reshape(H, 1).astype(jnp.float32)
    th2, a_s, a_t = _pre(source_h, target_h, W, b_lin.reshape(1, H),
                         w1, w2, b_att.reshape(1, 1))
    src2d = edge_list[0].astype(jnp.int32).reshape(NROWS, EB)
    tgt2d = edge_list[1].astype(jnp.int32).reshape(NROWS, EB)
    return _sc_edge(src2d, tgt2d, a_s.reshape(N), a_t.reshape(N), th2,
                    bias.astype(jnp.float32).reshape(NC, HW))


# speedup vs baseline: 1.2189x; 1.1418x over previous
import functools

import jax
import jax.numpy as jnp
from jax import lax
from jax.experimental import pallas as pl
from jax.experimental.pallas import tpu as pltpu
from jax.experimental.pallas import tpu_sc as plsc

N = 10000
E = 320000
D = 128
H = 128

NC = 2
NS = 16
LANES = 16
HW = H // NC

EB = 64
NROWS = E // EB
NROWS_PAD = 5120
R1T = NROWS_PAD // NS
NP = 10240
NSEG = NP // NS
HT = 624


def _pre_body(s_ref, t_ref, w_ref, bl_ref, w1_ref, w2_ref, ba_ref,
              th_ref, as_ref, at_ref):
    w = w_ref[...]
    dn = (((1,), (1,)), ((), ()))
    sh = lax.dot_general(s_ref[...], w, dn,
                         preferred_element_type=jnp.float32) + bl_ref[...]
    th = lax.dot_general(t_ref[...], w, dn,
                         preferred_element_type=jnp.float32) + bl_ref[...]
    th_ref[0] = th[:, :HW]
    th_ref[1] = th[:, HW:]
    as_ref[...] = sh @ w1_ref[...] + ba_ref[0, 0]
    at_ref[...] = th @ w2_ref[...]


def _pre(source_h, target_h, W, b_lin, w1, w2, b_att):
    blk = 1000
    grid = N // blk
    return pl.pallas_call(
        _pre_body,
        grid=(grid,),
        in_specs=[
            pl.BlockSpec((blk, D), lambda i: (i, 0)),
            pl.BlockSpec((blk, D), lambda i: (i, 0)),
            pl.BlockSpec((H, D), lambda i: (0, 0)),
            pl.BlockSpec((1, H), lambda i: (0, 0)),
            pl.BlockSpec((H, 1), lambda i: (0, 0)),
            pl.BlockSpec((H, 1), lambda i: (0, 0)),
            pl.BlockSpec((1, 1), lambda i: (0, 0)),
        ],
        out_specs=[
            pl.BlockSpec((2, blk, HW), lambda i: (0, i, 0)),
            pl.BlockSpec((blk, 1), lambda i: (i, 0)),
            pl.BlockSpec((blk, 1), lambda i: (i, 0)),
        ],
        out_shape=[
            jax.ShapeDtypeStruct((2, N, HW), jnp.float32),
            jax.ShapeDtypeStruct((N, 1), jnp.float32),
            jax.ShapeDtypeStruct((N, 1), jnp.float32),
        ],
    )(source_h, target_h, W, b_lin, w1, w2, b_att)


def _post_body(p_ref, b_ref, o_ref):
    o_ref[:, :HW] = p_ref[0] + b_ref[:, :HW]
    o_ref[:, HW:] = p_ref[1] + b_ref[:, HW:]


def _post(partials, bias):
    blk = 1000
    return pl.pallas_call(
        _post_body,
        grid=(N // blk,),
        in_specs=[
            pl.BlockSpec((2, blk, HW), lambda i: (0, i, 0)),
            pl.BlockSpec((1, H), lambda i: (0, 0)),
        ],
        out_specs=pl.BlockSpec((blk, H), lambda i: (i, 0)),
        out_shape=jax.ShapeDtypeStruct((N, H), jnp.float32),
    )(partials, bias)


def _edge_exp(a, b):
    x = a + b
    t = jnp.exp(-2.0 * jnp.abs(x))
    th = (1.0 - t) / (1.0 + t)
    e = jnp.where(x < 0.0, -th, th)
    return jnp.exp(e)


def _sc_body(src_hbm, tgt_hbm, as_hbm, at_hbm, th2_hbm, bias_hbm, out_hbm,
             asv, atv, dtile, srcv, tgtv, rows, biasv, didx,
             dacc, hp, gs0, gs1, gs2, gs3, ss0, ss1, ss2, ss3):
    gsem = [gs0, gs1, gs2, gs3]
    ssem = [ss0, ss1, ss2, ss3]
    c = lax.axis_index("c")
    s = lax.axis_index("s")
    thc = th2_hbm.at[c]

    pltpu.sync_copy(as_hbm, asv)
    pltpu.sync_copy(at_hbm, atv)
    pltpu.sync_copy(bias_hbm.at[c], biasv)

    zero = jnp.zeros((LANES,), jnp.float32)
    lanes_iota = lax.iota(jnp.int32, LANES)

    def zero_dtile(i, _):
        dtile[i] = zero
        return 0

    lax.fori_loop(0, NP // LANES, zero_dtile, 0)
    for k in range(NP // LANES // EB):
        for j in range(EB // LANES):
            didx[k, pl.ds(j * LANES, LANES)] = (
                lanes_iota + (k * EB + j * LANES))

    off = s * (NSEG // LANES)
    pltpu.sync_copy(dtile.at[pl.ds(0, NSEG // LANES)],
                    dacc.at[pl.ds(off, NSEG // LANES)])

    def bias_rows(i, _):
        for j in range(HW // LANES):
            sl = pl.ds(j * LANES, LANES)
            rows[0, i, sl] = biasv[sl]
        return 0

    lax.fori_loop(0, EB, bias_rows, 0)
    hbase = s * HT
    for k in range(HT // EB):
        pltpu.sync_copy(rows.at[0], hp.at[pl.ds(hbase + k * EB, EB)])
    rem = HT - (HT // EB) * EB
    pltpu.sync_copy(rows.at[0, pl.ds(0, rem)],
                    hp.at[pl.ds(hbase + (HT // EB) * EB, rem)])

    @pl.when(s == NS - 1)
    def _():
        pltpu.sync_copy(rows.at[0, pl.ds(0, 16)],
                        hp.at[pl.ds(NS * HT, 16)])

    cnt = jnp.minimum(R1T, NROWS - s * R1T)

    @pl.when(s < NS - 1)
    def _():
        pltpu.sync_copy(src_hbm.at[pl.ds(s * R1T, R1T)], srcv)
        pltpu.sync_copy(tgt_hbm.at[pl.ds(s * R1T, R1T)], tgtv)

    @pl.when(s == NS - 1)
    def _():
        lastn = NROWS - (NS - 1) * R1T
        pltpu.sync_copy(src_hbm.at[pl.ds((NS - 1) * R1T, lastn)],
                        srcv.at[pl.ds(0, lastn)])
        pltpu.sync_copy(tgt_hbm.at[pl.ds((NS - 1) * R1T, lastn)],
                        tgtv.at[pl.ds(0, lastn)])

    for b in range(4):
        pltpu.async_copy(thc.at[tgtv.at[b]], rows.at[b], gsem[b])

    def p1_row(r, _):
        for j in range(EB // LANES):
            s16 = srcv[r, pl.ds(j * LANES, LANES)]
            t16 = tgtv[r, pl.ds(j * LANES, LANES)]
            ex = _edge_exp(plsc.load_gather(asv, [s16]),
                           plsc.load_gather(atv, [t16]))
            plsc.addupdate_scatter(
                dtile,
                [lax.shift_right_logical(s16, 4),
                 lax.bitwise_and(s16, 15)], ex)
        return 0

    lax.fori_loop(0, cnt, p1_row, 0)

    plsc.subcore_barrier()
    for k in range(NP // LANES // EB):
        pltpu.sync_copy(dtile.at[pl.ds(k * EB, EB)],
                        dacc.at[didx.at[k]], add=True)
    plsc.subcore_barrier()

    pltpu.sync_copy(dacc, dtile)

    def issue(g, buf):
        pltpu.async_copy(thc.at[tgtv.at[g]], rows.at[buf], gsem[buf])

    def wait_gather(buf):
        pltpu.make_async_copy(thc.at[tgtv.at[0]], rows.at[buf],
                              gsem[buf]).wait()

    def scatter(g, buf):
        pltpu.async_copy(rows.at[buf], hp.at[srcv.at[g]], ssem[buf],
                         add=True)

    def wait_scatter(buf):
        pltpu.make_async_copy(rows.at[buf], hp.at[srcv.at[0]],
                              ssem[buf]).wait()

    def attn_chunk(g, base):
        s16 = srcv[g, pl.ds(base, LANES)]
        t16 = tgtv[g, pl.ds(base, LANES)]
        ex = _edge_exp(plsc.load_gather(asv, [s16]),
                       plsc.load_gather(atv, [t16]))
        d = plsc.load_gather(
            dtile,
            [lax.shift_right_logical(s16, 4),
             lax.bitwise_and(s16, 15)])
        return ex / d

    def consume(g, buf):
        def chunk(jc, attn16):
            nbase = jnp.where(jc == EB // LANES - 1, 0, (jc + 1) * LANES)
            nxt = attn_chunk(g, nbase)
            base = jc * LANES
            for l in range(LANES):
                a = attn16[l]
                i = base + l
                for j in range(HW // LANES):
                    sl = pl.ds(j * LANES, LANES)
                    rows[buf, i, sl] = rows[buf, i, sl] * a
            return nxt

        lax.fori_loop(0, EB // LANES, chunk, attn_chunk(g, 0))

    def p2_quad(i, _):
        for b in range(4):
            g = 4 * i + b
            wait_gather(b)
            consume(g, b)
            sc = (b + 3) % 4
            gp = g - 1

            @pl.when(gp >= 0)
            def _():
                wait_scatter(sc)

                @pl.when(gp + 4 < cnt)
                def _():
                    issue(gp + 4, sc)

            scatter(g, b)
        return 0

    lax.fori_loop(0, cnt // 4, p2_quad, 0)
    wait_scatter(3)

    plsc.subcore_barrier()
    col = c * HW
    pltpu.sync_copy(hp.at[pl.ds(hbase, HT)],
                    out_hbm.at[pl.ds(hbase, HT), pl.ds(col, HW)])

    @pl.when(s == NS - 1)
    def _():
        pltpu.sync_copy(hp.at[pl.ds(NS * HT, 16)],
                        out_hbm.at[pl.ds(NS * HT, 16), pl.ds(col, HW)])


_sc_edge = functools.partial(
    pl.kernel,
    out_type=jax.ShapeDtypeStruct((N, H), jnp.float32),
    mesh=plsc.VectorSubcoreMesh(core_axis_name="c", subcore_axis_name="s"),
    compiler_params=pltpu.CompilerParams(needs_layout_passes=False,
                                         use_tc_tiling_on_sc=False),
    scratch_types=[
        pltpu.VMEM((N,), jnp.float32),
        pltpu.VMEM((N,), jnp.float32),
        pltpu.VMEM((NP // LANES, LANES), jnp.float32),
        pltpu.VMEM((R1T, EB), jnp.int32),
        pltpu.VMEM((R1T, EB), jnp.int32),
        pltpu.VMEM((4, EB, HW), jnp.float32),
        pltpu.VMEM((HW,), jnp.float32),
        pltpu.VMEM((NP // LANES // EB, EB), jnp.int32),
        pltpu.VMEM_SHARED((NP // LANES, LANES), jnp.float32),
        pltpu.VMEM_SHARED((N, HW), jnp.float32),
        pltpu.SemaphoreType.DMA,
        pltpu.SemaphoreType.DMA,
        pltpu.SemaphoreType.DMA,
        pltpu.SemaphoreType.DMA,
        pltpu.SemaphoreType.DMA,
        pltpu.SemaphoreType.DMA,
        pltpu.SemaphoreType.DMA,
        pltpu.SemaphoreType.DMA,
    ],
)(_sc_body)


def kernel(source_h, target_h, edge_list, W, b_lin, W_att, b_att, bias):
    w1 = W_att[0, :H].reshape(H, 1).astype(jnp.float32)
    w2 = W_att[0, H:].reshape(H, 1).astype(jnp.float32)
    th2, a_s, a_t = _pre(source_h, target_h, W, b_lin.reshape(1, H),
                         w1, w2, b_att.reshape(1, 1))
    src2d = edge_list[0].astype(jnp.int32).reshape(NROWS, EB)
    tgt2d = edge_list[1].astype(jnp.int32).reshape(NROWS, EB)
    return _sc_edge(src2d, tgt2d, a_s.reshape(N), a_t.reshape(N), th2,
                    bias.astype(jnp.float32).reshape(NC, HW))
